# baseline (speedup 1.0000x reference)
import functools

import jax
import jax.numpy as jnp
from jax import lax
from jax.experimental import pallas as pl
from jax.experimental.pallas import tpu as pltpu
from jax.experimental.pallas import tpu_sc as plsc

N = 10000
P = 10240
D = 128
H1 = 128
H2 = 64
E = 320000
NC = 2
NS = 16
NW = NC * NS
CH = 128
NCHUNK = 80
E_PAD = NW * NCHUNK * CH
RPT = P // NS
DEG_W = 16

_MESH = plsc.VectorSubcoreMesh(
    core_axis_name="c", subcore_axis_name="s", num_cores=NC, num_subcores=NS)
_INTERP = False


@functools.partial(
    pl.kernel,
    out_type=jax.ShapeDtypeStruct((NC, P, DEG_W), jnp.float32),
    mesh=_MESH,
    scratch_types=[
        pltpu.VMEM((5, NCHUNK // 5, CH), jnp.int32),
        pltpu.VMEM((CH, DEG_W), jnp.float32),
        pltpu.VMEM((16, DEG_W), jnp.float32),
        pltpu.VMEM_SHARED((P, DEG_W), jnp.float32),
        pltpu.SemaphoreType.DMA,
    ],
    interpret=_INTERP,
)
def _deg_kernel(dst_hbm, deg_out, dst_v, ones_v, z_v, deg_sh, sem):
    c = lax.axis_index("c")
    s = lax.axis_index("s")
    wid = c * NS + s
    J = NCHUNK // 5
    zv = jnp.zeros((16,), jnp.float32)
    for r in range(16):
        z_v[r, :] = zv
    lane = lax.iota(jnp.int32, 16)
    onev = jnp.where(lane == 0, 1.0, 0.0).astype(jnp.float32)
    for r in range(CH):
        ones_v[r, :] = onev
    base = s * RPT
    def zero_body(t, carry):
        pltpu.sync_copy(z_v, deg_sh.at[pl.ds(base + 16 * t, 16)])
        return carry
    lax.fori_loop(0, RPT // 16, zero_body, 0)
    for g in range(5):
        pltpu.async_copy(dst_hbm.at[wid, pl.ds(g * J, J)], dst_v.at[g],
                         sem).wait()
    plsc.subcore_barrier()
    def body(g, carry):
        def inner(j, carry2):
            pltpu.sync_copy(ones_v, deg_sh.at[dst_v.at[g, j]], add=True)
            return carry2
        lax.fori_loop(0, J, inner, 0)
        return carry
    lax.fori_loop(0, 5, body, 0)
    plsc.subcore_barrier()
    pltpu.sync_copy(deg_sh.at[pl.ds(base, RPT)], deg_out.at[c, pl.ds(base, RPT)])


G = 16
NGRP = NCHUNK // G


@functools.partial(
    pl.kernel,
    out_type=jax.ShapeDtypeStruct((NC, P, D), jnp.float32),
    mesh=_MESH,
    scratch_types=[
        pltpu.VMEM((2, G, CH), jnp.int32),
        pltpu.VMEM((2, G, CH), jnp.int32),
        pltpu.VMEM((2, CH, D), jnp.float32),
        pltpu.VMEM((16, D), jnp.float32),
        pltpu.VMEM_SHARED((P, D), jnp.float32),
        pltpu.SemaphoreType.DMA,
        pltpu.SemaphoreType.DMA,
        pltpu.SemaphoreType.DMA((2,)),
        pltpu.SemaphoreType.DMA((2,)),
    ],
    interpret=_INTERP,
)
def _scatter_kernel(y_hbm, src_hbm, dst_hbm, acc_out,
                    src_v, dst_v, rows_v, z_v, acc_sh, isem_a, isem_b, gsem,
                    ssem):
    c = lax.axis_index("c")
    s = lax.axis_index("s")
    wid = c * NS + s
    zv = jnp.zeros((16,), jnp.float32)
    for r in range(16):
        for k in range(D // 16):
            z_v[r, pl.ds(16 * k, 16)] = zv
    base = s * RPT
    def zero_body(t, carry):
        pltpu.sync_copy(z_v, acc_sh.at[pl.ds(base + 16 * t, 16)])
        return carry
    lax.fori_loop(0, RPT // 16, zero_body, 0)
    pltpu.async_copy(src_hbm.at[wid, pl.ds(0, G)], src_v.at[0], isem_a).wait()
    pltpu.async_copy(dst_hbm.at[wid, pl.ds(0, G)], dst_v.at[0], isem_b).wait()
    plsc.subcore_barrier()

    def step(j, carry):
        g = lax.div(j, G)
        gslot = lax.rem(g, 2)
        p = lax.rem(j, 2)
        @pl.when(jnp.logical_and(j < NCHUNK,
                                 jnp.logical_and(lax.rem(j, G) == 0, g > 0)))
        def _():
            pltpu.make_async_copy(src_hbm.at[wid, pl.ds(g * G, G)],
                                  src_v.at[gslot], isem_a).wait()
            pltpu.make_async_copy(dst_hbm.at[wid, pl.ds(g * G, G)],
                                  dst_v.at[gslot], isem_b).wait()
        @pl.when(j == 0)
        def _():
            pltpu.async_copy(y_hbm.at[src_v.at[0, 0]], rows_v.at[0],
                             gsem.at[0])
        @pl.when(jnp.logical_and(j >= 1, j <= NCHUNK))
        def _():
            jm = j - 1
            gm = lax.rem(lax.div(jm, G), 2)
            q = lax.rem(jm, 2)
            pltpu.make_async_copy(y_hbm.at[src_v.at[gm, lax.rem(jm, G)]],
                                  rows_v.at[q], gsem.at[q]).wait()
            scat = pltpu.async_copy(rows_v.at[q],
                                    acc_sh.at[dst_v.at[gm, lax.rem(jm, G)]],
                                    ssem.at[q], add=True)
            @pl.when(j < NCHUNK)
            def _():
                pltpu.async_copy(y_hbm.at[src_v.at[gslot, lax.rem(j, G)]],
                                 rows_v.at[p], gsem.at[p])
            scat.wait()
        @pl.when(jnp.logical_and(j < NCHUNK,
                                 jnp.logical_and(lax.rem(j, G) == 2,
                                                 g + 1 < NGRP)))
        def _():
            pltpu.async_copy(src_hbm.at[wid, pl.ds((g + 1) * G, G)],
                             src_v.at[1 - gslot], isem_a)
            pltpu.async_copy(dst_hbm.at[wid, pl.ds((g + 1) * G, G)],
                             dst_v.at[1 - gslot], isem_b)
        return carry
    lax.fori_loop(0, NCHUNK + 1, step, 0)
    plsc.subcore_barrier()
    pltpu.sync_copy(acc_sh.at[pl.ds(base, RPT)], acc_out.at[c, pl.ds(base, RPT)])


_BLK = 2000


def _xw_body(x_ref, w_ref, deg_ref, y_ref, dinv_ref):
    deg = deg_ref[0, :, 0:1] + deg_ref[1, :, 0:1] + 1.0
    dinv = lax.rsqrt(deg)
    xw = jnp.dot(x_ref[...], w_ref[...], preferred_element_type=jnp.float32)
    y_ref[...] = xw * dinv
    dinv_ref[...] = dinv


def _xw_call(x, w, deg_parts):
    return pl.pallas_call(
        _xw_body,
        grid=(N // _BLK,),
        in_specs=[
            pl.BlockSpec((_BLK, D), lambda r: (r, 0)),
            pl.BlockSpec((D, D), lambda r: (0, 0)),
            pl.BlockSpec((NC, _BLK, DEG_W), lambda r: (0, r, 0)),
        ],
        out_specs=[
            pl.BlockSpec((_BLK, D), lambda r: (r, 0)),
            pl.BlockSpec((_BLK, 1), lambda r: (r, 0)),
        ],
        out_shape=[
            jax.ShapeDtypeStruct((N, D), jnp.float32),
            jax.ShapeDtypeStruct((N, 1), jnp.float32),
        ],
    )(x, w, deg_parts)


def _head_body(acc_ref, y_ref, dinv_ref, bconv_ref,
               w1_ref, b1_ref, w2_ref, b2_ref, w3_ref, b3_ref,
               out_ref, hacc):
    r = pl.program_id(0)
    @pl.when(r == 0)
    def _():
        hacc[...] = jnp.zeros_like(hacc)
    rows = (acc_ref[0] + acc_ref[1] + y_ref[...]) * dinv_ref[...]
    rows = jnp.maximum(rows + bconv_ref[...], 0.0)
    psum = jnp.sum(rows, axis=0, keepdims=True)
    hacc[...] += jnp.broadcast_to(psum, (8, D))
    @pl.when(r == N // _BLK - 1)
    def _():
        h = hacc[...]
        h1 = jnp.maximum(
            jnp.dot(h, w1_ref[...], preferred_element_type=jnp.float32)
            + b1_ref[...], 0.0)
        h2 = jnp.maximum(
            jnp.dot(h1, w2_ref[...], preferred_element_type=jnp.float32)
            + b2_ref[...], 0.0)
        h3 = jnp.dot(h2, w3_ref[...], preferred_element_type=jnp.float32)
        out_ref[...] = h3[0:1, :] + b3_ref[...]


def _head_call(acc_parts, y, dinv, b_conv, W1, b1, W2, b2, W3, b3):
    return pl.pallas_call(
        _head_body,
        grid=(N // _BLK,),
        in_specs=[
            pl.BlockSpec((NC, _BLK, D), lambda r: (0, r, 0)),
            pl.BlockSpec((_BLK, D), lambda r: (r, 0)),
            pl.BlockSpec((_BLK, 1), lambda r: (r, 0)),
            pl.BlockSpec((1, D), lambda r: (0, 0)),
            pl.BlockSpec((H1, H1), lambda r: (0, 0)),
            pl.BlockSpec((1, H1), lambda r: (0, 0)),
            pl.BlockSpec((H1, H2), lambda r: (0, 0)),
            pl.BlockSpec((1, H2), lambda r: (0, 0)),
            pl.BlockSpec((H2, 1), lambda r: (0, 0)),
            pl.BlockSpec((1, 1), lambda r: (0, 0)),
        ],
        out_specs=pl.BlockSpec((1, 1), lambda r: (0, 0)),
        out_shape=jax.ShapeDtypeStruct((1, 1), jnp.float32),
        scratch_shapes=[pltpu.VMEM((8, D), jnp.float32)],
    )(acc_parts, y, dinv, b_conv, W1, b1, W2, b2, W3, b3)


def kernel(x, edge_index, W_conv, b_conv, W1, b1, W2, b2, W3, b3):
    ei = edge_index.astype(jnp.int32)
    pad = E_PAD - E
    ar = jnp.arange(pad, dtype=jnp.int32)
    pad_src = (ar * 97) % N
    pad_dst = N + ar % (P - N)
    src = jnp.concatenate([ei[0], pad_src]).reshape(NW, NCHUNK, CH)
    dst = jnp.concatenate([ei[1], pad_dst]).reshape(NW, NCHUNK, CH)

    deg_parts = _deg_kernel(dst)
    y, dinv = _xw_call(x, W_conv, deg_parts)
    acc_parts = _scatter_kernel(y, src, dst)
    out = _head_call(acc_parts, y, dinv,
                     b_conv.reshape(1, D), W1, b1.reshape(1, H1),
                     W2, b2.reshape(1, H2), W3, b3.reshape(1, 1))
    return out.reshape(1)


# speedup vs baseline: 1.1045x; 1.1045x over previous
import functools

import jax
import jax.numpy as jnp
from jax import lax
from jax.experimental import pallas as pl
from jax.experimental.pallas import tpu as pltpu
from jax.experimental.pallas import tpu_sc as plsc

N = 10000
P = 10240
D = 128
H1 = 128
H2 = 64
E = 320000
NC = 2
NS = 16
NW = NC * NS
CH = 128
NCHUNK = 80
E_PAD = NW * NCHUNK * CH
RPT = P // NS
DEG_W = 16

_MESH = plsc.VectorSubcoreMesh(
    core_axis_name="c", subcore_axis_name="s", num_cores=NC, num_subcores=NS)
_INTERP = False


@functools.partial(
    pl.kernel,
    out_type=jax.ShapeDtypeStruct((NC, P, DEG_W), jnp.float32),
    mesh=_MESH,
    scratch_types=[
        pltpu.VMEM((5, NCHUNK // 5, CH), jnp.int32),
        pltpu.VMEM((CH, DEG_W), jnp.float32),
        pltpu.VMEM((16, DEG_W), jnp.float32),
        pltpu.VMEM_SHARED((P, DEG_W), jnp.float32),
        pltpu.SemaphoreType.DMA,
    ],
    interpret=_INTERP,
)
def _deg_kernel(dst_hbm, deg_out, dst_v, ones_v, z_v, deg_sh, sem):
    c = lax.axis_index("c")
    s = lax.axis_index("s")
    wid = c * NS + s
    J = NCHUNK // 5
    zv = jnp.zeros((16,), jnp.float32)
    for r in range(16):
        z_v[r, :] = zv
    lane = lax.iota(jnp.int32, 16)
    onev = jnp.where(lane == 0, 1.0, 0.0).astype(jnp.float32)
    for r in range(CH):
        ones_v[r, :] = onev
    base = s * RPT
    def zero_body(t, carry):
        pltpu.async_copy(z_v, deg_sh.at[pl.ds(base + 16 * t, 16)], sem)
        return carry
    lax.fori_loop(0, RPT // 16, zero_body, 0)
    def zero_drain(t, carry):
        pltpu.make_async_copy(z_v, deg_sh.at[pl.ds(base + 16 * t, 16)],
                              sem).wait()
        return carry
    lax.fori_loop(0, RPT // 16, zero_drain, 0)
    for g in range(5):
        pltpu.async_copy(dst_hbm.at[wid, pl.ds(g * J, J)], dst_v.at[g],
                         sem).wait()
    plsc.subcore_barrier()
    def body(g, carry):
        def inner(j, carry2):
            pltpu.sync_copy(ones_v, deg_sh.at[dst_v.at[g, j]], add=True)
            return carry2
        lax.fori_loop(0, J, inner, 0)
        return carry
    lax.fori_loop(0, 5, body, 0)
    plsc.subcore_barrier()
    pltpu.sync_copy(deg_sh.at[pl.ds(base, RPT)], deg_out.at[c, pl.ds(base, RPT)])


G = 16
NGRP = NCHUNK // G


@functools.partial(
    pl.kernel,
    out_type=jax.ShapeDtypeStruct((NC, P, D), jnp.float32),
    mesh=_MESH,
    scratch_types=[
        pltpu.VMEM((2, G, CH), jnp.int32),
        pltpu.VMEM((2, G, CH), jnp.int32),
        pltpu.VMEM((2, CH, D), jnp.float32),
        pltpu.VMEM((16, D), jnp.float32),
        pltpu.VMEM_SHARED((P, D), jnp.float32),
        pltpu.SemaphoreType.DMA,
        pltpu.SemaphoreType.DMA,
        pltpu.SemaphoreType.DMA((2,)),
        pltpu.SemaphoreType.DMA,
    ],
    interpret=_INTERP,
)
def _scatter_kernel(y_hbm, src_hbm, dst_hbm, acc_out,
                    src_v, dst_v, rows_v, z_v, acc_sh, isem_a, isem_b, gsem,
                    zsem):
    c = lax.axis_index("c")
    s = lax.axis_index("s")
    wid = c * NS + s
    zv = jnp.zeros((16,), jnp.float32)
    for r in range(16):
        for k in range(D // 16):
            z_v[r, pl.ds(16 * k, 16)] = zv
    base = s * RPT
    def zero_body(t, carry):
        pltpu.async_copy(z_v, acc_sh.at[pl.ds(base + 16 * t, 16)], zsem)
        return carry
    lax.fori_loop(0, RPT // 16, zero_body, 0)
    def zero_drain(t, carry):
        pltpu.make_async_copy(z_v, acc_sh.at[pl.ds(base + 16 * t, 16)],
                              zsem).wait()
        return carry
    lax.fori_loop(0, RPT // 16, zero_drain, 0)
    pltpu.async_copy(src_hbm.at[wid, pl.ds(0, G)], src_v.at[0], isem_a).wait()
    pltpu.async_copy(dst_hbm.at[wid, pl.ds(0, G)], dst_v.at[0], isem_b).wait()
    plsc.subcore_barrier()

    def group(g, carry):
        gp = lax.rem(g, 2)
        gq = 1 - gp
        @pl.when(g + 1 < NGRP)
        def _():
            pltpu.async_copy(src_hbm.at[wid, pl.ds((g + 1) * G, G)],
                             src_v.at[gq], isem_a)
            pltpu.async_copy(dst_hbm.at[wid, pl.ds((g + 1) * G, G)],
                             dst_v.at[gq], isem_b)
        pltpu.async_copy(y_hbm.at[src_v.at[gp, 0]], rows_v.at[0], gsem.at[0])
        def body(j, carry2):
            p = lax.rem(j, 2)
            q = 1 - p
            @pl.when(j + 1 < G)
            def _():
                pltpu.async_copy(y_hbm.at[src_v.at[gp, j + 1]], rows_v.at[q],
                                 gsem.at[q])
            pltpu.make_async_copy(y_hbm.at[src_v.at[gp, j]], rows_v.at[p],
                                  gsem.at[p]).wait()
            pltpu.sync_copy(rows_v.at[p], acc_sh.at[dst_v.at[gp, j]], add=True)
            return carry2
        lax.fori_loop(0, G, body, 0)
        @pl.when(g + 1 < NGRP)
        def _():
            pltpu.make_async_copy(src_hbm.at[wid, pl.ds((g + 1) * G, G)],
                                  src_v.at[gq], isem_a).wait()
            pltpu.make_async_copy(dst_hbm.at[wid, pl.ds((g + 1) * G, G)],
                                  dst_v.at[gq], isem_b).wait()
        return carry
    lax.fori_loop(0, NGRP, group, 0)
    plsc.subcore_barrier()
    pltpu.sync_copy(acc_sh.at[pl.ds(base, RPT)], acc_out.at[c, pl.ds(base, RPT)])


_BLK = 2000


def _mm_body(x_ref, w_ref, xw_ref):
    xw_ref[...] = jnp.dot(x_ref[...], w_ref[...],
                          preferred_element_type=jnp.float32)


def _mm_call(x, w):
    return pl.pallas_call(
        _mm_body,
        grid=(N // _BLK,),
        in_specs=[
            pl.BlockSpec((_BLK, D), lambda r: (r, 0)),
            pl.BlockSpec((D, D), lambda r: (0, 0)),
        ],
        out_specs=pl.BlockSpec((_BLK, D), lambda r: (r, 0)),
        out_shape=jax.ShapeDtypeStruct((N, D), jnp.float32),
    )(x, w)


def _xw_body(xw_ref, deg_ref, y_ref, dinv_ref):
    deg = deg_ref[0, :, 0:1] + deg_ref[1, :, 0:1] + 1.0
    dinv = lax.rsqrt(deg)
    y_ref[...] = xw_ref[...] * dinv
    dinv_ref[...] = dinv


def _xw_call(xw, deg_parts):
    return pl.pallas_call(
        _xw_body,
        grid=(N // _BLK,),
        in_specs=[
            pl.BlockSpec((_BLK, D), lambda r: (r, 0)),
            pl.BlockSpec((NC, _BLK, DEG_W), lambda r: (0, r, 0)),
        ],
        out_specs=[
            pl.BlockSpec((_BLK, D), lambda r: (r, 0)),
            pl.BlockSpec((_BLK, 1), lambda r: (r, 0)),
        ],
        out_shape=[
            jax.ShapeDtypeStruct((N, D), jnp.float32),
            jax.ShapeDtypeStruct((N, 1), jnp.float32),
        ],
    )(xw, deg_parts)


def _head_body(acc_ref, y_ref, dinv_ref, bconv_ref,
               w1_ref, b1_ref, w2_ref, b2_ref, w3_ref, b3_ref,
               out_ref, hacc):
    r = pl.program_id(0)
    @pl.when(r == 0)
    def _():
        hacc[...] = jnp.zeros_like(hacc)
    rows = (acc_ref[0] + acc_ref[1] + y_ref[...]) * dinv_ref[...]
    rows = jnp.maximum(rows + bconv_ref[...], 0.0)
    psum = jnp.sum(rows, axis=0, keepdims=True)
    hacc[...] += jnp.broadcast_to(psum, (8, D))
    @pl.when(r == N // _BLK - 1)
    def _():
        h = hacc[...]
        h1 = jnp.maximum(
            jnp.dot(h, w1_ref[...], preferred_element_type=jnp.float32)
            + b1_ref[...], 0.0)
        h2 = jnp.maximum(
            jnp.dot(h1, w2_ref[...], preferred_element_type=jnp.float32)
            + b2_ref[...], 0.0)
        h3 = jnp.dot(h2, w3_ref[...], preferred_element_type=jnp.float32)
        out_ref[...] = h3[0:1, :] + b3_ref[...]


def _head_call(acc_parts, y, dinv, b_conv, W1, b1, W2, b2, W3, b3):
    return pl.pallas_call(
        _head_body,
        grid=(N // _BLK,),
        in_specs=[
            pl.BlockSpec((NC, _BLK, D), lambda r: (0, r, 0)),
            pl.BlockSpec((_BLK, D), lambda r: (r, 0)),
            pl.BlockSpec((_BLK, 1), lambda r: (r, 0)),
            pl.BlockSpec((1, D), lambda r: (0, 0)),
            pl.BlockSpec((H1, H1), lambda r: (0, 0)),
            pl.BlockSpec((1, H1), lambda r: (0, 0)),
            pl.BlockSpec((H1, H2), lambda r: (0, 0)),
            pl.BlockSpec((1, H2), lambda r: (0, 0)),
            pl.BlockSpec((H2, 1), lambda r: (0, 0)),
            pl.BlockSpec((1, 1), lambda r: (0, 0)),
        ],
        out_specs=pl.BlockSpec((1, 1), lambda r: (0, 0)),
        out_shape=jax.ShapeDtypeStruct((1, 1), jnp.float32),
        scratch_shapes=[pltpu.VMEM((8, D), jnp.float32)],
    )(acc_parts, y, dinv, b_conv, W1, b1, W2, b2, W3, b3)


def kernel(x, edge_index, W_conv, b_conv, W1, b1, W2, b2, W3, b3):
    ei = edge_index.astype(jnp.int32)
    pad = E_PAD - E
    ar = jnp.arange(pad, dtype=jnp.int32)
    pad_src = (ar * 97) % N
    pad_dst = N + ar % (P - N)
    src = jnp.concatenate([ei[0], pad_src]).reshape(NW, NCHUNK, CH)
    dst = jnp.concatenate([ei[1], pad_dst]).reshape(NW, NCHUNK, CH)

    xw = _mm_call(x, W_conv)
    deg_parts = _deg_kernel(dst)
    y, dinv = _xw_call(xw, deg_parts)
    acc_parts = _scatter_kernel(y, src, dst)
    out = _head_call(acc_parts, y, dinv,
                     b_conv.reshape(1, D), W1, b1.reshape(1, H1),
                     W2, b2.reshape(1, H2), W3, b3.reshape(1, 1))
    return out.reshape(1)
